# baseline (speedup 1.0000x reference)
import functools

import jax
import jax.numpy as jnp
from jax import lax
from jax.experimental import pallas as pl
from jax.experimental.pallas import tpu as pltpu
from jax.experimental.pallas import tpu_sc as plsc

N = 8192
D = 1024
H = 1024
OD = 128
HD = 64
E = 8

NC = 2
NS = 16
NW = NC * NS
CH = N // NW

BLK = 1024
PN = N + E * BLK
NBLK = PN // BLK
NBLK_PAD = 16


def _mesh():
    return plsc.VectorSubcoreMesh(core_axis_name="c", subcore_axis_name="s",
                                  num_cores=NC)


def _sc_params():
    return pltpu.CompilerParams(needs_layout_passes=False)


def _wid():
    return lax.axis_index("s") * NC + lax.axis_index("c")


def _iota16():
    return lax.iota(jnp.int32, 16)


def _count_body(ids_hbm, counts_hbm, ids_v, cv):
    w = _wid()
    pltpu.sync_copy(ids_hbm.at[pl.ds(w * CH, CH)], ids_v)
    cnt = [jnp.int32(0)] * E
    for v in range(CH // 16):
        idv = ids_v[pl.ds(v * 16, 16)]
        for e in range(E):
            cnt[e] = cnt[e] + jnp.sum((idv == e).astype(jnp.int32))
    it = _iota16()
    vec = jnp.zeros((16,), jnp.int32)
    for e in range(E):
        vec = jnp.where(it == e, cnt[e], vec)
    cv[...] = vec
    pltpu.sync_copy(cv, counts_hbm.at[w])


@functools.cache
def _count_call():
    return pl.kernel(
        _count_body,
        out_type=jax.ShapeDtypeStruct((NW, 16), jnp.int32),
        mesh=_mesh(),
        compiler_params=_sc_params(),
        scratch_types=[
            pltpu.VMEM((CH,), jnp.int32),
            pltpu.VMEM((16,), jnp.int32),
        ],
    )


def _route_body(ids_hbm, x_hbm, counts_hbm, pos_hbm, xs_hbm, bexp_hbm,
                ids_v, lv, cb, posbuf, xbuf, bexp_v,
                s_ld0, s_ld1, s_sc0, s_sc1):
    c = lax.axis_index("c")
    t = lax.axis_index("s")
    it = _iota16()
    p = t * NC + c
    row0 = p * CH
    pltpu.sync_copy(ids_hbm.at[pl.ds(row0, CH)], ids_v)
    pltpu.sync_copy(counts_hbm, lv)

    tot = jnp.zeros((16,), jnp.int32)
    pre = jnp.zeros((16,), jnp.int32)
    for w2 in range(NW):
        row = lv[w2, :]
        tot = tot + row
        pre = pre + row * (jnp.int32(w2) < p).astype(jnp.int32)
    pad = ((tot + (BLK - 1)) // BLK) * BLK
    incl = plsc.cumsum(pad)
    off = incl - pad
    base_vec = off + pre
    base = [jnp.sum(jnp.where(it == e, base_vec, 0)) for e in range(E)]

    @pl.when((t == 0) & (c == 0))
    def _():
        total_used = jnp.sum(pad)
        for g in range(NBLK_PAD // 16):
            bstart = (it + g * 16) * BLK
            acc = jnp.zeros((16,), jnp.int32)
            for e in range(E):
                end_e = jnp.sum(jnp.where(it == e, incl, 0))
                acc = acc + (bstart >= end_e).astype(jnp.int32)
            bexp_v[pl.ds(g * 16, 16)] = jnp.where(bstart >= total_used, -1, acc)
        pltpu.sync_copy(bexp_v, bexp_hbm)

    NCK = CH // 32

    def pos_chunk(j, buf):
        for k in range(2):
            idv = ids_v[pl.ds(j * 32 + k * 16, 16)]
            posv = jnp.zeros((16,), jnp.int32)
            for e in range(E):
                m = idv == e
                cs = plsc.cumsum(m.astype(jnp.int32))
                posv = jnp.where(m, base[e] + cs - 1, posv)
                base[e] = base[e] + jnp.sum(m.astype(jnp.int32))
            cb[buf, pl.ds(k * 16, 16)] = posv
            posbuf[pl.ds(j * 32 + k * 16, 16)] = posv

    s_ld = [s_ld0, s_ld1]
    s_sc = [s_sc0, s_sc1]
    ld = [None, None]
    sc = [None, None]
    pos_chunk(0, 0)
    ld[0] = pltpu.async_copy(x_hbm.at[pl.ds(row0, 32)], xbuf.at[0], s_ld[0])
    for j in range(NCK):
        cur = j % 2
        nxt = 1 - cur
        if j + 1 < NCK:
            if sc[nxt] is not None:
                sc[nxt].wait()
                sc[nxt] = None
            pos_chunk(j + 1, nxt)
            ld[nxt] = pltpu.async_copy(
                x_hbm.at[pl.ds(row0 + (j + 1) * 32, 32)], xbuf.at[nxt], s_ld[nxt])
        ld[cur].wait()
        sc[cur] = pltpu.async_copy(xbuf.at[cur], xs_hbm.at[cb.at[cur]], s_sc[cur])
    pltpu.sync_copy(posbuf, pos_hbm.at[pl.ds(row0, CH)])
    for b in range(2):
        if sc[b] is not None:
            sc[b].wait()


@functools.cache
def _route_call():
    return pl.kernel(
        _route_body,
        out_type=[
            jax.ShapeDtypeStruct((N,), jnp.int32),
            jax.ShapeDtypeStruct((PN, D), jnp.float32),
            jax.ShapeDtypeStruct((NBLK_PAD,), jnp.int32),
        ],
        mesh=_mesh(),
        compiler_params=_sc_params(),
        scratch_types=[
            pltpu.VMEM((CH,), jnp.int32),
            pltpu.VMEM((NW, 16), jnp.int32),
            pltpu.VMEM((2, 32), jnp.int32),
            pltpu.VMEM((CH,), jnp.int32),
            pltpu.VMEM((2, 32, D), jnp.float32),
            pltpu.VMEM((NBLK_PAD,), jnp.int32),
            pltpu.SemaphoreType.DMA,
            pltpu.SemaphoreType.DMA,
            pltpu.SemaphoreType.DMA,
            pltpu.SemaphoreType.DMA,
        ],
    )


def _emap(v):
    return jnp.where(v < 0, E - 1, v).astype(jnp.int32)


def _gmm_body(be_ref, x_ref, w1_ref, b1_ref, w2_ref, b2_ref, o_ref):
    b = pl.program_id(0)

    @pl.when(be_ref[b] >= 0)
    def _():
        xb = x_ref[...].astype(jnp.bfloat16)
        h = jnp.maximum(
            jnp.dot(xb, w1_ref[0], preferred_element_type=jnp.float32)
            + b1_ref[0], 0.0)
        o = jnp.dot(h.astype(jnp.bfloat16), w2_ref[0],
                    preferred_element_type=jnp.float32) + b2_ref[0]
        sq = jnp.concatenate([o[:, :HD], o[:, HD:] * o[:, HD:]], axis=1)
        o_ref[...] = sq


_gmm_call = pl.pallas_call(
    _gmm_body,
    grid_spec=pltpu.PrefetchScalarGridSpec(
        num_scalar_prefetch=1,
        grid=(NBLK,),
        in_specs=[
            pl.BlockSpec((BLK, D), lambda b, be: (b, 0)),
            pl.BlockSpec((1, D, H), lambda b, be: (_emap(be[b]), 0, 0)),
            pl.BlockSpec((1, 1, H), lambda b, be: (_emap(be[b]), 0, 0)),
            pl.BlockSpec((1, H, OD), lambda b, be: (_emap(be[b]), 0, 0)),
            pl.BlockSpec((1, 1, OD), lambda b, be: (_emap(be[b]), 0, 0)),
        ],
        out_specs=[
            pl.BlockSpec((BLK, OD), lambda b, be: (b, 0)),
        ],
    ),
    out_shape=[
        jax.ShapeDtypeStruct((PN, OD), jnp.float32),
    ],
)


def _gather_body(pos_hbm, o_hbm, head_hbm, hvar_hbm, idxbuf, rows, sem):
    w = _wid()
    for j in range(CH // 128):
        base = w * CH + j * 128
        pltpu.sync_copy(pos_hbm.at[pl.ds(base, 128)], idxbuf)
        pltpu.async_copy(o_hbm.at[idxbuf], rows, sem).wait()
        pltpu.sync_copy(rows.at[:, pl.ds(0, HD)], head_hbm.at[pl.ds(base, 128)])
        pltpu.sync_copy(rows.at[:, pl.ds(HD, HD)], hvar_hbm.at[pl.ds(base, 128)])


@functools.cache
def _gather_call():
    return pl.kernel(
        _gather_body,
        out_type=[
            jax.ShapeDtypeStruct((N, HD), jnp.float32),
            jax.ShapeDtypeStruct((N, HD), jnp.float32),
        ],
        mesh=_mesh(),
        compiler_params=pltpu.CompilerParams(needs_layout_passes=False,
                                             use_tc_tiling_on_sc=False),
        scratch_types=[
            pltpu.VMEM((128,), jnp.int32),
            pltpu.VMEM((128, OD), jnp.float32),
            pltpu.SemaphoreType.DMA,
        ],
    )


def kernel(x_graph, W1, b1, W2, b2, dataset_ids):
    ids = dataset_ids.astype(jnp.int32)
    counts = _count_call()(ids)
    pos, xs, bexp = _route_call()(ids, x_graph, counts)
    (o,) = _gmm_call(bexp, xs, W1.astype(jnp.bfloat16), b1.reshape(E, 1, H),
                     W2.astype(jnp.bfloat16), b2.reshape(E, 1, OD))
    head, headvar = _gather_call()(pos, o)
    return head, headvar


# speedup vs baseline: 1.1056x; 1.1056x over previous
import functools

import jax
import jax.numpy as jnp
from jax import lax
from jax.experimental import pallas as pl
from jax.experimental.pallas import tpu as pltpu
from jax.experimental.pallas import tpu_sc as plsc

N = 8192
D = 1024
H = 1024
OD = 128
HD = 64
E = 8

NC = 2
NS = 16
NW = NC * NS
CH = N // NW

BLK = 1024
PN = N + E * BLK
NBLK = PN // BLK
NBLK_PAD = 16


def _mesh():
    return plsc.VectorSubcoreMesh(core_axis_name="c", subcore_axis_name="s",
                                  num_cores=NC)


def _sc_params():
    return pltpu.CompilerParams(needs_layout_passes=False)


def _wid():
    return lax.axis_index("s") * NC + lax.axis_index("c")


def _iota16():
    return lax.iota(jnp.int32, 16)


def _count_body(ids_hbm, counts_hbm, ids_v, cv):
    w = _wid()
    pltpu.sync_copy(ids_hbm.at[pl.ds(w * CH, CH)], ids_v)
    cnt = [jnp.int32(0)] * E
    for v in range(CH // 16):
        idv = ids_v[pl.ds(v * 16, 16)]
        for e in range(E):
            cnt[e] = cnt[e] + jnp.sum((idv == e).astype(jnp.int32))
    it = _iota16()
    vec = jnp.zeros((16,), jnp.int32)
    for e in range(E):
        vec = jnp.where(it == e, cnt[e], vec)
    cv[...] = vec
    pltpu.sync_copy(cv, counts_hbm.at[w])


@functools.cache
def _count_call():
    return pl.kernel(
        _count_body,
        out_type=jax.ShapeDtypeStruct((NW, 16), jnp.int32),
        mesh=_mesh(),
        compiler_params=_sc_params(),
        scratch_types=[
            pltpu.VMEM((CH,), jnp.int32),
            pltpu.VMEM((16,), jnp.int32),
        ],
    )


def _route_body(ids_hbm, x_hbm, counts_hbm, pos_hbm, xs_hbm, bexp_hbm,
                ids_v, lv, cb, posbuf, xbuf, bexp_v,
                s_ld0, s_ld1, s_sc0, s_sc1):
    c = lax.axis_index("c")
    t = lax.axis_index("s")
    it = _iota16()
    p = t * NC + c
    row0 = p * CH
    pltpu.sync_copy(ids_hbm.at[pl.ds(row0, CH)], ids_v)
    pltpu.sync_copy(counts_hbm, lv)

    tot = jnp.zeros((16,), jnp.int32)
    pre = jnp.zeros((16,), jnp.int32)
    for w2 in range(NW):
        row = lv[w2, :]
        tot = tot + row
        pre = pre + row * (jnp.int32(w2) < p).astype(jnp.int32)
    pad = ((tot + (BLK - 1)) // BLK) * BLK
    incl = plsc.cumsum(pad)
    off = incl - pad
    base_vec = off + pre
    base = [jnp.sum(jnp.where(it == e, base_vec, 0)) for e in range(E)]

    @pl.when((t == 0) & (c == 0))
    def _():
        total_used = jnp.sum(pad)
        for g in range(NBLK_PAD // 16):
            bstart = (it + g * 16) * BLK
            acc = jnp.zeros((16,), jnp.int32)
            for e in range(E):
                end_e = jnp.sum(jnp.where(it == e, incl, 0))
                acc = acc + (bstart >= end_e).astype(jnp.int32)
            bexp_v[pl.ds(g * 16, 16)] = jnp.where(bstart >= total_used, -1, acc)
        pltpu.sync_copy(bexp_v, bexp_hbm)

    NCK = CH // 32

    def pos_chunk(j, buf):
        for k in range(2):
            idv = ids_v[pl.ds(j * 32 + k * 16, 16)]
            posv = jnp.zeros((16,), jnp.int32)
            for e in range(E):
                m = idv == e
                cs = plsc.cumsum(m.astype(jnp.int32))
                posv = jnp.where(m, base[e] + cs - 1, posv)
                base[e] = base[e] + jnp.sum(m.astype(jnp.int32))
            cb[buf, pl.ds(k * 16, 16)] = posv
            posbuf[pl.ds(j * 32 + k * 16, 16)] = posv

    s_ld = [s_ld0, s_ld1]
    s_sc = [s_sc0, s_sc1]
    ld = [None, None]
    sc = [None, None]
    pos_chunk(0, 0)
    ld[0] = pltpu.async_copy(x_hbm.at[pl.ds(row0, 32)], xbuf.at[0], s_ld[0])
    for j in range(NCK):
        cur = j % 2
        nxt = 1 - cur
        if j + 1 < NCK:
            if sc[nxt] is not None:
                sc[nxt].wait()
                sc[nxt] = None
            pos_chunk(j + 1, nxt)
            ld[nxt] = pltpu.async_copy(
                x_hbm.at[pl.ds(row0 + (j + 1) * 32, 32)], xbuf.at[nxt], s_ld[nxt])
        ld[cur].wait()
        sc[cur] = pltpu.async_copy(xbuf.at[cur], xs_hbm.at[cb.at[cur]], s_sc[cur])
    pltpu.sync_copy(posbuf, pos_hbm.at[pl.ds(row0, CH)])
    for b in range(2):
        if sc[b] is not None:
            sc[b].wait()


@functools.cache
def _route_call():
    return pl.kernel(
        _route_body,
        out_type=[
            jax.ShapeDtypeStruct((N,), jnp.int32),
            jax.ShapeDtypeStruct((PN, D), jnp.float32),
            jax.ShapeDtypeStruct((NBLK_PAD,), jnp.int32),
        ],
        mesh=_mesh(),
        compiler_params=_sc_params(),
        scratch_types=[
            pltpu.VMEM((CH,), jnp.int32),
            pltpu.VMEM((NW, 16), jnp.int32),
            pltpu.VMEM((2, 32), jnp.int32),
            pltpu.VMEM((CH,), jnp.int32),
            pltpu.VMEM((2, 32, D), jnp.float32),
            pltpu.VMEM((NBLK_PAD,), jnp.int32),
            pltpu.SemaphoreType.DMA,
            pltpu.SemaphoreType.DMA,
            pltpu.SemaphoreType.DMA,
            pltpu.SemaphoreType.DMA,
        ],
    )


def _emap(v):
    return jnp.where(v < 0, E - 1, v).astype(jnp.int32)


def _gmm_body(be_ref, x_ref, w1_ref, b1_ref, w2_ref, b2_ref, o_ref):
    b = pl.program_id(0)

    @pl.when(be_ref[b] >= 0)
    def _():
        h = jnp.maximum(
            jnp.dot(x_ref[...], w1_ref[0], preferred_element_type=jnp.float32)
            + b1_ref[0], 0.0)
        o = jnp.dot(h, w2_ref[0], preferred_element_type=jnp.float32) + b2_ref[0]
        sq = jnp.concatenate([o[:, :HD], o[:, HD:] * o[:, HD:]], axis=1)
        o_ref[...] = sq


_gmm_call = pl.pallas_call(
    _gmm_body,
    grid_spec=pltpu.PrefetchScalarGridSpec(
        num_scalar_prefetch=1,
        grid=(NBLK,),
        in_specs=[
            pl.BlockSpec((BLK, D), lambda b, be: (b, 0)),
            pl.BlockSpec((1, D, H), lambda b, be: (_emap(be[b]), 0, 0)),
            pl.BlockSpec((1, 1, H), lambda b, be: (_emap(be[b]), 0, 0)),
            pl.BlockSpec((1, H, OD), lambda b, be: (_emap(be[b]), 0, 0)),
            pl.BlockSpec((1, 1, OD), lambda b, be: (_emap(be[b]), 0, 0)),
        ],
        out_specs=[
            pl.BlockSpec((BLK, OD), lambda b, be: (b, 0)),
        ],
    ),
    out_shape=[
        jax.ShapeDtypeStruct((PN, OD), jnp.float32),
    ],
)


def _gather_body(pos_hbm, o_hbm, comb_hbm, idxbuf, rows, sem):
    w = _wid()
    for j in range(CH // 128):
        base = w * CH + j * 128
        pltpu.sync_copy(pos_hbm.at[pl.ds(base, 128)], idxbuf)
        pltpu.async_copy(o_hbm.at[idxbuf], rows, sem).wait()
        pltpu.sync_copy(rows, comb_hbm.at[pl.ds(base, 128)])


@functools.cache
def _gather_call():
    return pl.kernel(
        _gather_body,
        out_type=jax.ShapeDtypeStruct((N, OD), jnp.float32),
        mesh=_mesh(),
        compiler_params=_sc_params(),
        scratch_types=[
            pltpu.VMEM((128,), jnp.int32),
            pltpu.VMEM((128, OD), jnp.float32),
            pltpu.SemaphoreType.DMA,
        ],
    )


SBLK = 1024


def _split_body(c_ref, ht_ref, vt_ref):
    ot = c_ref[...].T
    ht_ref[...] = ot[:HD, :]
    vt_ref[...] = ot[HD:, :]


_split_call = pl.pallas_call(
    _split_body,
    grid=(N // SBLK,),
    in_specs=[pl.BlockSpec((SBLK, OD), lambda b: (b, 0))],
    out_specs=[pl.BlockSpec((HD, SBLK), lambda b: (0, b)),
               pl.BlockSpec((HD, SBLK), lambda b: (0, b))],
    out_shape=[jax.ShapeDtypeStruct((HD, N), jnp.float32),
               jax.ShapeDtypeStruct((HD, N), jnp.float32)],
)


def kernel(x_graph, W1, b1, W2, b2, dataset_ids):
    ids = dataset_ids.astype(jnp.int32)
    counts = _count_call()(ids)
    pos, xs, bexp = _route_call()(ids, x_graph, counts)
    (o,) = _gmm_call(bexp, xs, W1, b1.reshape(E, 1, H), W2,
                     b2.reshape(E, 1, OD))
    comb = _gather_call()(pos, o)
    headT, varT = _split_call(comb)
    return headT.T, varT.T


# speedup vs baseline: 1.1429x; 1.0337x over previous
import functools

import jax
import jax.numpy as jnp
from jax import lax
from jax.experimental import pallas as pl
from jax.experimental.pallas import tpu as pltpu
from jax.experimental.pallas import tpu_sc as plsc

N = 8192
D = 1024
H = 1024
OD = 128
HD = 64
E = 8

NC = 2
NS = 16
NW = NC * NS
CH = N // NW

BLK = 1024
PN = N + E * BLK
NBLK = PN // BLK
NBLK_PAD = 16


def _mesh():
    return plsc.VectorSubcoreMesh(core_axis_name="c", subcore_axis_name="s",
                                  num_cores=NC)


def _sc_params():
    return pltpu.CompilerParams(needs_layout_passes=False)


def _wid():
    return lax.axis_index("s") * NC + lax.axis_index("c")


def _iota16():
    return lax.iota(jnp.int32, 16)


def _count_body(ids_hbm, counts_hbm, ids_v, cv):
    w = _wid()
    pltpu.sync_copy(ids_hbm.at[pl.ds(w * CH, CH)], ids_v)
    cnt = [jnp.int32(0)] * E
    for v in range(CH // 16):
        idv = ids_v[pl.ds(v * 16, 16)]
        for e in range(E):
            cnt[e] = cnt[e] + jnp.sum((idv == e).astype(jnp.int32))
    it = _iota16()
    vec = jnp.zeros((16,), jnp.int32)
    for e in range(E):
        vec = jnp.where(it == e, cnt[e], vec)
    cv[...] = vec
    pltpu.sync_copy(cv, counts_hbm.at[w])


@functools.cache
def _count_call():
    return pl.kernel(
        _count_body,
        out_type=jax.ShapeDtypeStruct((NW, 16), jnp.int32),
        mesh=_mesh(),
        compiler_params=_sc_params(),
        scratch_types=[
            pltpu.VMEM((CH,), jnp.int32),
            pltpu.VMEM((16,), jnp.int32),
        ],
    )


def _route_body(ids_hbm, x_hbm, counts_hbm, pos_hbm, xs_hbm, bexp_hbm,
                ids_v, lv, cb, posbuf, xbuf, bexp_v,
                s_ld0, s_ld1, s_sc0, s_sc1):
    c = lax.axis_index("c")
    t = lax.axis_index("s")
    it = _iota16()
    p = t * NC + c
    row0 = p * CH
    pltpu.sync_copy(ids_hbm.at[pl.ds(row0, CH)], ids_v)
    pltpu.sync_copy(counts_hbm, lv)

    tot = jnp.zeros((16,), jnp.int32)
    pre = jnp.zeros((16,), jnp.int32)
    for w2 in range(NW):
        row = lv[w2, :]
        tot = tot + row
        pre = pre + row * (jnp.int32(w2) < p).astype(jnp.int32)
    pad = ((tot + (BLK - 1)) // BLK) * BLK
    incl = plsc.cumsum(pad)
    off = incl - pad
    base_vec = off + pre
    base = [jnp.sum(jnp.where(it == e, base_vec, 0)) for e in range(E)]

    @pl.when((t == 0) & (c == 0))
    def _():
        total_used = jnp.sum(pad)
        for g in range(NBLK_PAD // 16):
            bstart = (it + g * 16) * BLK
            acc = jnp.zeros((16,), jnp.int32)
            for e in range(E):
                end_e = jnp.sum(jnp.where(it == e, incl, 0))
                acc = acc + (bstart >= end_e).astype(jnp.int32)
            bexp_v[pl.ds(g * 16, 16)] = jnp.where(bstart >= total_used, -1, acc)
        pltpu.sync_copy(bexp_v, bexp_hbm)

    NCK = CH // 32

    def pos_chunk(j, buf):
        for k in range(2):
            idv = ids_v[pl.ds(j * 32 + k * 16, 16)]
            posv = jnp.zeros((16,), jnp.int32)
            for e in range(E):
                m = idv == e
                cs = plsc.cumsum(m.astype(jnp.int32))
                posv = jnp.where(m, base[e] + cs - 1, posv)
                base[e] = base[e] + jnp.sum(m.astype(jnp.int32))
            cb[buf, pl.ds(k * 16, 16)] = posv
            posbuf[pl.ds(j * 32 + k * 16, 16)] = posv

    s_ld = [s_ld0, s_ld1]
    s_sc = [s_sc0, s_sc1]
    ld = [None, None]
    sc = [None, None]
    pos_chunk(0, 0)
    ld[0] = pltpu.async_copy(x_hbm.at[pl.ds(row0, 32)], xbuf.at[0], s_ld[0])
    for j in range(NCK):
        cur = j % 2
        nxt = 1 - cur
        if j + 1 < NCK:
            if sc[nxt] is not None:
                sc[nxt].wait()
                sc[nxt] = None
            pos_chunk(j + 1, nxt)
            ld[nxt] = pltpu.async_copy(
                x_hbm.at[pl.ds(row0 + (j + 1) * 32, 32)], xbuf.at[nxt], s_ld[nxt])
        ld[cur].wait()
        sc[cur] = pltpu.async_copy(xbuf.at[cur], xs_hbm.at[cb.at[cur]], s_sc[cur])
    pltpu.sync_copy(posbuf, pos_hbm.at[pl.ds(row0, CH)])
    for b in range(2):
        if sc[b] is not None:
            sc[b].wait()


@functools.cache
def _route_call():
    return pl.kernel(
        _route_body,
        out_type=[
            jax.ShapeDtypeStruct((N,), jnp.int32),
            jax.ShapeDtypeStruct((PN, D), jnp.float32),
            jax.ShapeDtypeStruct((NBLK_PAD,), jnp.int32),
        ],
        mesh=_mesh(),
        compiler_params=_sc_params(),
        scratch_types=[
            pltpu.VMEM((CH,), jnp.int32),
            pltpu.VMEM((NW, 16), jnp.int32),
            pltpu.VMEM((2, 32), jnp.int32),
            pltpu.VMEM((CH,), jnp.int32),
            pltpu.VMEM((2, 32, D), jnp.float32),
            pltpu.VMEM((NBLK_PAD,), jnp.int32),
            pltpu.SemaphoreType.DMA,
            pltpu.SemaphoreType.DMA,
            pltpu.SemaphoreType.DMA,
            pltpu.SemaphoreType.DMA,
        ],
    )


def _emap(v):
    return jnp.where(v < 0, E - 1, v).astype(jnp.int32)


def _gmm_body(be_ref, x_ref, w1_ref, b1_ref, w2_ref, b2_ref, o_ref):
    b = pl.program_id(0)

    @pl.when(be_ref[b] >= 0)
    def _():
        h = jnp.maximum(
            jnp.dot(x_ref[...], w1_ref[0], preferred_element_type=jnp.float32)
            + b1_ref[0], 0.0)
        o = jnp.dot(h, w2_ref[0], preferred_element_type=jnp.float32) + b2_ref[0]
        sq = jnp.concatenate([o[:, :HD], o[:, HD:] * o[:, HD:]], axis=1)
        o_ref[...] = sq


_gmm_call = pl.pallas_call(
    _gmm_body,
    grid_spec=pltpu.PrefetchScalarGridSpec(
        num_scalar_prefetch=1,
        grid=(NBLK,),
        in_specs=[
            pl.BlockSpec((BLK, D), lambda b, be: (jnp.where(be[b] < 0, 0, b), 0)),
            pl.BlockSpec((1, D, H), lambda b, be: (_emap(be[b]), 0, 0)),
            pl.BlockSpec((1, 1, H), lambda b, be: (_emap(be[b]), 0, 0)),
            pl.BlockSpec((1, H, OD), lambda b, be: (_emap(be[b]), 0, 0)),
            pl.BlockSpec((1, 1, OD), lambda b, be: (_emap(be[b]), 0, 0)),
        ],
        out_specs=[
            pl.BlockSpec((BLK, OD), lambda b, be: (b, 0)),
        ],
    ),
    out_shape=[
        jax.ShapeDtypeStruct((PN, OD), jnp.float32),
    ],
)


def _gather_body(pos_hbm, o_hbm, comb_hbm, idxbuf, rows, sem):
    w = _wid()
    for j in range(CH // 128):
        base = w * CH + j * 128
        pltpu.sync_copy(pos_hbm.at[pl.ds(base, 128)], idxbuf)
        pltpu.async_copy(o_hbm.at[idxbuf], rows, sem).wait()
        pltpu.sync_copy(rows, comb_hbm.at[pl.ds(base, 128)])


@functools.cache
def _gather_call():
    return pl.kernel(
        _gather_body,
        out_type=jax.ShapeDtypeStruct((N, OD), jnp.float32),
        mesh=_mesh(),
        compiler_params=_sc_params(),
        scratch_types=[
            pltpu.VMEM((128,), jnp.int32),
            pltpu.VMEM((128, OD), jnp.float32),
            pltpu.SemaphoreType.DMA,
        ],
    )


SBLK = 1024


def _split_body(c_ref, ht_ref, vt_ref):
    ot = c_ref[...].T
    ht_ref[...] = ot[:HD, :]
    vt_ref[...] = ot[HD:, :]


_split_call = pl.pallas_call(
    _split_body,
    grid=(N // SBLK,),
    in_specs=[pl.BlockSpec((SBLK, OD), lambda b: (b, 0))],
    out_specs=[pl.BlockSpec((HD, SBLK), lambda b: (0, b)),
               pl.BlockSpec((HD, SBLK), lambda b: (0, b))],
    out_shape=[jax.ShapeDtypeStruct((HD, N), jnp.float32),
               jax.ShapeDtypeStruct((HD, N), jnp.float32)],
)


def kernel(x_graph, W1, b1, W2, b2, dataset_ids):
    ids = dataset_ids.astype(jnp.int32)
    counts = _count_call()(ids)
    pos, xs, bexp = _route_call()(ids, x_graph, counts)
    (o,) = _gmm_call(bexp, xs, W1, b1.reshape(E, 1, H), W2,
                     b2.reshape(E, 1, OD))
    comb = _gather_call()(pos, o)
    headT, varT = _split_call(comb)
    return headT.T, varT.T


# speedup vs baseline: 1.1439x; 1.0009x over previous
import functools

import jax
import jax.numpy as jnp
from jax import lax
from jax.experimental import pallas as pl
from jax.experimental.pallas import tpu as pltpu
from jax.experimental.pallas import tpu_sc as plsc

N = 8192
D = 1024
H = 1024
OD = 128
HD = 64
E = 8

NC = 2
NS = 16
NW = NC * NS
CH = N // NW

BLK = 1024
PN = N + E * BLK
NBLK = PN // BLK
NBLK_PAD = 16


def _mesh():
    return plsc.VectorSubcoreMesh(core_axis_name="c", subcore_axis_name="s",
                                  num_cores=NC)


def _sc_params():
    return pltpu.CompilerParams(needs_layout_passes=False)


def _wid():
    return lax.axis_index("s") * NC + lax.axis_index("c")


def _iota16():
    return lax.iota(jnp.int32, 16)


def _count_body(ids_hbm, counts_hbm, ids_v, cv):
    w = _wid()
    pltpu.sync_copy(ids_hbm.at[pl.ds(w * CH, CH)], ids_v)
    cnt = [jnp.int32(0)] * E
    for v in range(CH // 16):
        idv = ids_v[pl.ds(v * 16, 16)]
        for e in range(E):
            cnt[e] = cnt[e] + jnp.sum((idv == e).astype(jnp.int32))
    it = _iota16()
    vec = jnp.zeros((16,), jnp.int32)
    for e in range(E):
        vec = jnp.where(it == e, cnt[e], vec)
    cv[...] = vec
    pltpu.sync_copy(cv, counts_hbm.at[w])


@functools.cache
def _count_call():
    return pl.kernel(
        _count_body,
        out_type=jax.ShapeDtypeStruct((NW, 16), jnp.int32),
        mesh=_mesh(),
        compiler_params=_sc_params(),
        scratch_types=[
            pltpu.VMEM((CH,), jnp.int32),
            pltpu.VMEM((16,), jnp.int32),
        ],
    )


def _route_body(ids_hbm, x_hbm, counts_hbm, pos_hbm, xs_hbm, bexp_hbm,
                ids_v, lv, cb, posbuf, xbuf, bexp_v,
                s_ld0, s_ld1, s_sc0, s_sc1):
    c = lax.axis_index("c")
    t = lax.axis_index("s")
    it = _iota16()
    p = t * NC + c
    row0 = p * CH
    pltpu.sync_copy(ids_hbm.at[pl.ds(row0, CH)], ids_v)
    pltpu.sync_copy(counts_hbm, lv)

    tot = jnp.zeros((16,), jnp.int32)
    pre = jnp.zeros((16,), jnp.int32)
    for w2 in range(NW):
        row = lv[w2, :]
        tot = tot + row
        pre = pre + row * (jnp.int32(w2) < p).astype(jnp.int32)
    pad = ((tot + (BLK - 1)) // BLK) * BLK
    incl = plsc.cumsum(pad)
    off = incl - pad
    base_vec = off + pre
    base = [jnp.sum(jnp.where(it == e, base_vec, 0)) for e in range(E)]

    @pl.when((t == 0) & (c == 0))
    def _():
        total_used = jnp.sum(pad)
        for g in range(NBLK_PAD // 16):
            bstart = (it + g * 16) * BLK
            acc = jnp.zeros((16,), jnp.int32)
            for e in range(E):
                end_e = jnp.sum(jnp.where(it == e, incl, 0))
                acc = acc + (bstart >= end_e).astype(jnp.int32)
            bexp_v[pl.ds(g * 16, 16)] = jnp.where(bstart >= total_used, -1, acc)
        pltpu.sync_copy(bexp_v, bexp_hbm)

    NCK = CH // 32

    def pos_chunk(j, buf):
        for k in range(2):
            idv = ids_v[pl.ds(j * 32 + k * 16, 16)]
            posv = jnp.zeros((16,), jnp.int32)
            for e in range(E):
                m = idv == e
                cs = plsc.cumsum(m.astype(jnp.int32))
                posv = jnp.where(m, base[e] + cs - 1, posv)
                base[e] = base[e] + jnp.sum(m.astype(jnp.int32))
            cb[buf, pl.ds(k * 16, 16)] = posv
            posbuf[pl.ds(j * 32 + k * 16, 16)] = posv

    s_ld = [s_ld0, s_ld1]
    s_sc = [s_sc0, s_sc1]
    ld = [None, None]
    sc = [None, None]
    pos_chunk(0, 0)
    ld[0] = pltpu.async_copy(x_hbm.at[pl.ds(row0, 32)], xbuf.at[0], s_ld[0])
    for j in range(NCK):
        cur = j % 2
        nxt = 1 - cur
        if j + 1 < NCK:
            if sc[nxt] is not None:
                sc[nxt].wait()
                sc[nxt] = None
            pos_chunk(j + 1, nxt)
            ld[nxt] = pltpu.async_copy(
                x_hbm.at[pl.ds(row0 + (j + 1) * 32, 32)], xbuf.at[nxt], s_ld[nxt])
        ld[cur].wait()
        sc[cur] = pltpu.async_copy(xbuf.at[cur], xs_hbm.at[cb.at[cur]], s_sc[cur])
    pltpu.sync_copy(posbuf, pos_hbm.at[pl.ds(row0, CH)])
    for b in range(2):
        if sc[b] is not None:
            sc[b].wait()


@functools.cache
def _route_call():
    return pl.kernel(
        _route_body,
        out_type=[
            jax.ShapeDtypeStruct((N,), jnp.int32),
            jax.ShapeDtypeStruct((PN, D), jnp.float32),
            jax.ShapeDtypeStruct((NBLK_PAD,), jnp.int32),
        ],
        mesh=_mesh(),
        compiler_params=_sc_params(),
        scratch_types=[
            pltpu.VMEM((CH,), jnp.int32),
            pltpu.VMEM((NW, 16), jnp.int32),
            pltpu.VMEM((2, 32), jnp.int32),
            pltpu.VMEM((CH,), jnp.int32),
            pltpu.VMEM((2, 32, D), jnp.float32),
            pltpu.VMEM((NBLK_PAD,), jnp.int32),
            pltpu.SemaphoreType.DMA,
            pltpu.SemaphoreType.DMA,
            pltpu.SemaphoreType.DMA,
            pltpu.SemaphoreType.DMA,
        ],
    )


def _emap(v):
    return jnp.where(v < 0, E - 1, v).astype(jnp.int32)


def _gmm_body(be_ref, x_ref, w1_ref, b1_ref, w2_ref, b2_ref, o_ref):
    b = pl.program_id(0)

    @pl.when(be_ref[b] >= 0)
    def _():
        h = jnp.maximum(
            jnp.dot(x_ref[...], w1_ref[0], preferred_element_type=jnp.float32,
                    precision=lax.Precision.DEFAULT)
            + b1_ref[0], 0.0)
        o = jnp.dot(h, w2_ref[0], preferred_element_type=jnp.float32,
                    precision=lax.Precision.DEFAULT) + b2_ref[0]
        sq = jnp.concatenate([o[:, :HD], o[:, HD:] * o[:, HD:]], axis=1)
        o_ref[...] = sq


_gmm_call = pl.pallas_call(
    _gmm_body,
    grid_spec=pltpu.PrefetchScalarGridSpec(
        num_scalar_prefetch=1,
        grid=(NBLK,),
        in_specs=[
            pl.BlockSpec((BLK, D), lambda b, be: (jnp.where(be[b] < 0, 0, b), 0)),
            pl.BlockSpec((1, D, H), lambda b, be: (_emap(be[b]), 0, 0)),
            pl.BlockSpec((1, 1, H), lambda b, be: (_emap(be[b]), 0, 0)),
            pl.BlockSpec((1, H, OD), lambda b, be: (_emap(be[b]), 0, 0)),
            pl.BlockSpec((1, 1, OD), lambda b, be: (_emap(be[b]), 0, 0)),
        ],
        out_specs=[
            pl.BlockSpec((BLK, OD), lambda b, be: (b, 0)),
        ],
    ),
    out_shape=[
        jax.ShapeDtypeStruct((PN, OD), jnp.float32),
    ],
)


def _gather_body(pos_hbm, o_hbm, comb_hbm, idxbuf, rows, sem):
    w = _wid()
    for j in range(CH // 128):
        base = w * CH + j * 128
        pltpu.sync_copy(pos_hbm.at[pl.ds(base, 128)], idxbuf)
        pltpu.async_copy(o_hbm.at[idxbuf], rows, sem).wait()
        pltpu.sync_copy(rows, comb_hbm.at[pl.ds(base, 128)])


@functools.cache
def _gather_call():
    return pl.kernel(
        _gather_body,
        out_type=jax.ShapeDtypeStruct((N, OD), jnp.float32),
        mesh=_mesh(),
        compiler_params=_sc_params(),
        scratch_types=[
            pltpu.VMEM((128,), jnp.int32),
            pltpu.VMEM((128, OD), jnp.float32),
            pltpu.SemaphoreType.DMA,
        ],
    )


SBLK = 1024


def _split_body(c_ref, ht_ref, vt_ref):
    ot = c_ref[...].T
    ht_ref[...] = ot[:HD, :]
    vt_ref[...] = ot[HD:, :]


_split_call = pl.pallas_call(
    _split_body,
    grid=(N // SBLK,),
    in_specs=[pl.BlockSpec((SBLK, OD), lambda b: (b, 0))],
    out_specs=[pl.BlockSpec((HD, SBLK), lambda b: (0, b)),
               pl.BlockSpec((HD, SBLK), lambda b: (0, b))],
    out_shape=[jax.ShapeDtypeStruct((HD, N), jnp.float32),
               jax.ShapeDtypeStruct((HD, N), jnp.float32)],
)


def kernel(x_graph, W1, b1, W2, b2, dataset_ids):
    ids = dataset_ids.astype(jnp.int32)
    counts = _count_call()(ids)
    pos, xs, bexp = _route_call()(ids, x_graph, counts)
    (o,) = _gmm_call(bexp, xs, W1, b1.reshape(E, 1, H), W2,
                     b2.reshape(E, 1, OD))
    comb = _gather_call()(pos, o)
    headT, varT = _split_call(comb)
    return headT.T, varT.T


# speedup vs baseline: 1.1568x; 1.0113x over previous
import functools

import jax
import jax.numpy as jnp
from jax import lax
from jax.experimental import pallas as pl
from jax.experimental.pallas import tpu as pltpu
from jax.experimental.pallas import tpu_sc as plsc

N = 8192
D = 1024
H = 1024
OD = 128
HD = 64
E = 8

NC = 2
NS = 16
NW = NC * NS
CH = N // NW

BLK = 1024
PN = N + E * BLK
NBLK = PN // BLK
NBLK_PAD = 16


def _mesh():
    return plsc.VectorSubcoreMesh(core_axis_name="c", subcore_axis_name="s",
                                  num_cores=NC)


def _sc_params():
    return pltpu.CompilerParams(needs_layout_passes=False)


def _wid():
    return lax.axis_index("s") * NC + lax.axis_index("c")


def _iota16():
    return lax.iota(jnp.int32, 16)


def _count_body(ids_hbm, counts_hbm, ids_v, cv):
    w = _wid()
    pltpu.sync_copy(ids_hbm.at[pl.ds(w * CH, CH)], ids_v)
    cnt = [jnp.int32(0)] * E
    for v in range(CH // 16):
        idv = ids_v[pl.ds(v * 16, 16)]
        for e in range(E):
            cnt[e] = cnt[e] + jnp.sum((idv == e).astype(jnp.int32))
    it = _iota16()
    vec = jnp.zeros((16,), jnp.int32)
    for e in range(E):
        vec = jnp.where(it == e, cnt[e], vec)
    cv[...] = vec
    pltpu.sync_copy(cv, counts_hbm.at[w])


@functools.cache
def _count_call():
    return pl.kernel(
        _count_body,
        out_type=jax.ShapeDtypeStruct((NW, 16), jnp.int32),
        mesh=_mesh(),
        compiler_params=_sc_params(),
        scratch_types=[
            pltpu.VMEM((CH,), jnp.int32),
            pltpu.VMEM((16,), jnp.int32),
        ],
    )


def _route_body(ids_hbm, x_hbm, counts_hbm, pos_hbm, xs_hbm, bexp_hbm,
                ids_v, lv, cb, posbuf, xbuf, bexp_v,
                s_ld0, s_ld1, s_sc0, s_sc1):
    c = lax.axis_index("c")
    t = lax.axis_index("s")
    it = _iota16()
    p = t * NC + c
    row0 = p * CH
    s_ld = [s_ld0, s_ld1]
    s_sc = [s_sc0, s_sc1]
    ld = [None, None]
    sc = [None, None]
    ld[0] = pltpu.async_copy(x_hbm.at[pl.ds(row0, 32)], xbuf.at[0], s_ld[0])
    pltpu.sync_copy(ids_hbm.at[pl.ds(row0, CH)], ids_v)
    pltpu.sync_copy(counts_hbm, lv)

    tot = jnp.zeros((16,), jnp.int32)
    pre = jnp.zeros((16,), jnp.int32)
    for w2 in range(NW):
        row = lv[w2, :]
        tot = tot + row
        pre = pre + row * (jnp.int32(w2) < p).astype(jnp.int32)
    pad = ((tot + (BLK - 1)) // BLK) * BLK
    incl = plsc.cumsum(pad)
    off = incl - pad
    base_vec = off + pre
    base = [jnp.sum(jnp.where(it == e, base_vec, 0)) for e in range(E)]

    @pl.when((t == 0) & (c == 0))
    def _():
        total_used = jnp.sum(pad)
        for g in range(NBLK_PAD // 16):
            bstart = (it + g * 16) * BLK
            acc = jnp.zeros((16,), jnp.int32)
            for e in range(E):
                end_e = jnp.sum(jnp.where(it == e, incl, 0))
                acc = acc + (bstart >= end_e).astype(jnp.int32)
            bexp_v[pl.ds(g * 16, 16)] = jnp.where(bstart >= total_used, -1, acc)
        pltpu.sync_copy(bexp_v, bexp_hbm)

    NCK = CH // 32

    def pos_chunk(j, buf):
        for k in range(2):
            idv = ids_v[pl.ds(j * 32 + k * 16, 16)]
            posv = jnp.zeros((16,), jnp.int32)
            for e in range(E):
                m = idv == e
                cs = plsc.cumsum(m.astype(jnp.int32))
                posv = jnp.where(m, base[e] + cs - 1, posv)
                base[e] = base[e] + jnp.sum(m.astype(jnp.int32))
            cb[buf, pl.ds(k * 16, 16)] = posv
            posbuf[pl.ds(j * 32 + k * 16, 16)] = posv

    pos_chunk(0, 0)
    for j in range(NCK):
        cur = j % 2
        nxt = 1 - cur
        if j + 1 < NCK:
            if sc[nxt] is not None:
                sc[nxt].wait()
                sc[nxt] = None
            ld[nxt] = pltpu.async_copy(
                x_hbm.at[pl.ds(row0 + (j + 1) * 32, 32)], xbuf.at[nxt], s_ld[nxt])
            pos_chunk(j + 1, nxt)
        ld[cur].wait()
        sc[cur] = pltpu.async_copy(xbuf.at[cur], xs_hbm.at[cb.at[cur]], s_sc[cur])
    pltpu.sync_copy(posbuf, pos_hbm.at[pl.ds(row0, CH)])
    for b in range(2):
        if sc[b] is not None:
            sc[b].wait()


@functools.cache
def _route_call():
    return pl.kernel(
        _route_body,
        out_type=[
            jax.ShapeDtypeStruct((N,), jnp.int32),
            jax.ShapeDtypeStruct((PN, D), jnp.float32),
            jax.ShapeDtypeStruct((NBLK_PAD,), jnp.int32),
        ],
        mesh=_mesh(),
        compiler_params=_sc_params(),
        scratch_types=[
            pltpu.VMEM((CH,), jnp.int32),
            pltpu.VMEM((NW, 16), jnp.int32),
            pltpu.VMEM((2, 32), jnp.int32),
            pltpu.VMEM((CH,), jnp.int32),
            pltpu.VMEM((2, 32, D), jnp.float32),
            pltpu.VMEM((NBLK_PAD,), jnp.int32),
            pltpu.SemaphoreType.DMA,
            pltpu.SemaphoreType.DMA,
            pltpu.SemaphoreType.DMA,
            pltpu.SemaphoreType.DMA,
        ],
    )


def _emap(v):
    return jnp.where(v < 0, E - 1, v).astype(jnp.int32)


def _gmm_body(be_ref, x_ref, w1_ref, b1_ref, w2_ref, b2_ref, o_ref):
    b = pl.program_id(0)

    @pl.when(be_ref[b] >= 0)
    def _():
        h = jnp.maximum(
            jnp.dot(x_ref[...], w1_ref[0], preferred_element_type=jnp.float32,
                    precision=lax.Precision.DEFAULT)
            + b1_ref[0], 0.0)
        o = jnp.dot(h, w2_ref[0], preferred_element_type=jnp.float32,
                    precision=lax.Precision.DEFAULT) + b2_ref[0]
        sq = jnp.concatenate([o[:, :HD], o[:, HD:] * o[:, HD:]], axis=1)
        o_ref[...] = sq


_gmm_call = pl.pallas_call(
    _gmm_body,
    grid_spec=pltpu.PrefetchScalarGridSpec(
        num_scalar_prefetch=1,
        grid=(NBLK,),
        in_specs=[
            pl.BlockSpec((BLK, D), lambda b, be: (jnp.where(be[b] < 0, 0, b), 0)),
            pl.BlockSpec((1, D, H), lambda b, be: (_emap(be[b]), 0, 0)),
            pl.BlockSpec((1, 1, H), lambda b, be: (_emap(be[b]), 0, 0)),
            pl.BlockSpec((1, H, OD), lambda b, be: (_emap(be[b]), 0, 0)),
            pl.BlockSpec((1, 1, OD), lambda b, be: (_emap(be[b]), 0, 0)),
        ],
        out_specs=[
            pl.BlockSpec((BLK, OD), lambda b, be: (b, 0)),
        ],
    ),
    out_shape=[
        jax.ShapeDtypeStruct((PN, OD), jnp.float32),
    ],
)


def _gather_body(pos_hbm, o_hbm, comb_hbm, idxbuf, rows, sem):
    w = _wid()
    for j in range(CH // 128):
        base = w * CH + j * 128
        pltpu.sync_copy(pos_hbm.at[pl.ds(base, 128)], idxbuf)
        pltpu.async_copy(o_hbm.at[idxbuf], rows, sem).wait()
        pltpu.sync_copy(rows, comb_hbm.at[pl.ds(base, 128)])


@functools.cache
def _gather_call():
    return pl.kernel(
        _gather_body,
        out_type=jax.ShapeDtypeStruct((N, OD), jnp.float32),
        mesh=_mesh(),
        compiler_params=_sc_params(),
        scratch_types=[
            pltpu.VMEM((128,), jnp.int32),
            pltpu.VMEM((128, OD), jnp.float32),
            pltpu.SemaphoreType.DMA,
        ],
    )


SBLK = 1024


def _split_body(c_ref, ht_ref, vt_ref):
    ot = c_ref[...].T
    ht_ref[...] = ot[:HD, :]
    vt_ref[...] = ot[HD:, :]


_split_call = pl.pallas_call(
    _split_body,
    grid=(N // SBLK,),
    in_specs=[pl.BlockSpec((SBLK, OD), lambda b: (b, 0))],
    out_specs=[pl.BlockSpec((HD, SBLK), lambda b: (0, b)),
               pl.BlockSpec((HD, SBLK), lambda b: (0, b))],
    out_shape=[jax.ShapeDtypeStruct((HD, N), jnp.float32),
               jax.ShapeDtypeStruct((HD, N), jnp.float32)],
)


def kernel(x_graph, W1, b1, W2, b2, dataset_ids):
    ids = dataset_ids.astype(jnp.int32)
    counts = _count_call()(ids)
    pos, xs, bexp = _route_call()(ids, x_graph, counts)
    (o,) = _gmm_call(bexp, xs, W1, b1.reshape(E, 1, H), W2,
                     b2.reshape(E, 1, OD))
    comb = _gather_call()(pos, o)
    headT, varT = _split_call(comb)
    return headT.T, varT.T
